# baseline (speedup 1.0000x reference)
import functools

import jax
import jax.numpy as jnp
from jax import lax
from jax.experimental import pallas as pl
from jax.experimental.pallas import tpu as pltpu
from jax.experimental.pallas import tpu_sc as plsc

_L = 16


def _sc_loss_partials(batch_T, gt_T, w_rep, b_rep, n_tiles, rows_per_tile):
    D, Bn = batch_T.shape
    groups = rows_per_tile // _L

    mesh = plsc.VectorSubcoreMesh(core_axis_name="c", subcore_axis_name="s")

    @functools.partial(
        pl.kernel,
        out_type=jax.ShapeDtypeStruct((n_tiles, _L), jnp.float32),
        mesh=mesh,
        compiler_params=pltpu.CompilerParams(needs_layout_passes=False),
        scratch_types=[
            pltpu.VMEM((D, rows_per_tile), jnp.float32),
            pltpu.VMEM((D, rows_per_tile), jnp.float32),
            pltpu.VMEM((D * D * _L,), jnp.float32),
            pltpu.VMEM((D * _L,), jnp.float32),
            pltpu.VMEM((D * _L,), jnp.float32),
            pltpu.VMEM((D * _L,), jnp.float32),
            pltpu.VMEM((_L,), jnp.float32),
        ],
    )
    def sc_kernel(bT_hbm, gT_hbm, wrep_hbm, brep_hbm, part_hbm,
                  bT_v, gT_v, wrep_v, brep_v, out_v, rgt_v, acc_v):
        num_cores = lax.axis_size("c")
        wid = lax.axis_index("s") * num_cores + lax.axis_index("c")
        base = wid * rows_per_tile

        pltpu.sync_copy(bT_hbm.at[:, pl.ds(base, rows_per_tile)], bT_v)
        pltpu.sync_copy(gT_hbm.at[:, pl.ds(base, rows_per_tile)], gT_v)
        pltpu.sync_copy(wrep_hbm, wrep_v)
        pltpu.sync_copy(brep_hbm, brep_v)

        def group_body(g, acc):
            g16 = g * _L

            o = [brep_v[pl.ds(j * _L, _L)] for j in range(D)]
            for d in range(D):
                xd = bT_v[d, pl.ds(g16, _L)]
                for j in range(D):
                    o[j] = o[j] + xd * wrep_v[pl.ds((d * D + j) * _L, _L)]
            for j in range(D):
                out_v[pl.ds(j * _L, _L)] = o[j]

            gcols = [gT_v[d, pl.ds(g16, _L)] for d in range(D)]
            rg = [jnp.full((_L,), float(j), jnp.float32) for j in range(D)]
            for a in range(D):
                for c in range(a + 1, D):
                    t = (gcols[c] < gcols[a]).astype(jnp.float32)
                    rg[a] = rg[a] + t
                    rg[c] = rg[c] - t
            for j in range(D):
                rgt_v[pl.ds(j * _L, _L)] = rg[j]

            o = [out_v[pl.ds(j * _L, _L)] for j in range(D)]
            ro = [jnp.full((_L,), float(j), jnp.float32) for j in range(D)]
            for a in range(D):
                for c in range(a + 1, D):
                    t = (o[c] > o[a]).astype(jnp.float32)
                    ro[a] = ro[a] + t
                    ro[c] = ro[c] - t

            for j in range(D):
                rgj = rgt_v[pl.ds(j * _L, _L)]
                w = jnp.exp(-rgj)
                dif = ro[j] - rgj - o[j]
                acc = acc + jnp.maximum(dif, 0.0) * w
            return acc

        acc = lax.fori_loop(0, groups, group_body,
                            jnp.zeros((_L,), jnp.float32))
        acc_v[...] = acc
        pltpu.sync_copy(acc_v, part_hbm.at[wid])

    return sc_kernel(batch_T, gt_T, w_rep, b_rep)


def kernel(batch, gt, W, b):
    Bn, D = batch.shape
    n_tiles = 32
    rows_per_tile = Bn // n_tiles
    batch_T = jnp.transpose(batch)
    gt_T = jnp.transpose(gt)
    w_rep = jnp.repeat(W.reshape(-1), _L)
    b_rep = jnp.repeat(b, _L)
    parts = _sc_loss_partials(batch_T, gt_T, w_rep, b_rep,
                              n_tiles, rows_per_tile)
    return jnp.sum(parts) * (1.0 / Bn)


# speedup vs baseline: 2.0839x; 2.0839x over previous
import functools

import jax
import jax.numpy as jnp
from jax import lax
from jax.experimental import pallas as pl
from jax.experimental.pallas import tpu as pltpu
from jax.experimental.pallas import tpu_sc as plsc

_L = 16


def _tc_linear_t(batch, gt, W, b):
    Bn, D = batch.shape
    blk = 2048
    grid = Bn // blk

    def body(batch_ref, W_ref, b_ref, out_ref):
        out_ref[...] = jnp.dot(batch_ref[...], W_ref[...],
                               preferred_element_type=jnp.float32) + b_ref[...]

    out = pl.pallas_call(
        body,
        grid=(grid,),
        in_specs=[
            pl.BlockSpec((blk, D), lambda i: (i, 0)),
            pl.BlockSpec((D, D), lambda i: (0, 0)),
            pl.BlockSpec((1, D), lambda i: (0, 0)),
        ],
        out_specs=pl.BlockSpec((blk, D), lambda i: (i, 0)),
        out_shape=jax.ShapeDtypeStruct((Bn, D), jnp.float32),
    )(batch, W, b.reshape(1, D))
    return jnp.transpose(out), jnp.transpose(gt)


def _sc_partials(out_T, gt_T, n_tiles, rows_per_tile):
    D, Bn = out_T.shape
    groups = rows_per_tile // _L

    mesh = plsc.VectorSubcoreMesh(core_axis_name="c", subcore_axis_name="s")

    @functools.partial(
        pl.kernel,
        out_type=jax.ShapeDtypeStruct((n_tiles, _L), jnp.float32),
        mesh=mesh,
        compiler_params=pltpu.CompilerParams(needs_layout_passes=False),
        scratch_types=[
            pltpu.VMEM((D, rows_per_tile), jnp.float32),
            pltpu.VMEM((D, rows_per_tile), jnp.float32),
            pltpu.VMEM((D * _L,), jnp.float32),
            pltpu.VMEM((_L,), jnp.float32),
        ],
    )
    def sc_kernel(outT_hbm, gtT_hbm, part_hbm, oT_v, gT_v, rgt_v, acc_v):
        num_cores = lax.axis_size("c")
        wid = lax.axis_index("s") * num_cores + lax.axis_index("c")
        base = wid * rows_per_tile

        pltpu.sync_copy(outT_hbm.at[:, pl.ds(base, rows_per_tile)], oT_v)
        pltpu.sync_copy(gtT_hbm.at[:, pl.ds(base, rows_per_tile)], gT_v)

        def group_body(g, acc):
            g16 = g * _L

            gcols = [gT_v[d, pl.ds(g16, _L)] for d in range(D)]
            rg = [jnp.full((_L,), float(j), jnp.float32) for j in range(D)]
            for a in range(D):
                for c in range(a + 1, D):
                    t = (gcols[c] < gcols[a]).astype(jnp.float32)
                    rg[a] = rg[a] + t
                    rg[c] = rg[c] - t
            for j in range(D):
                rgt_v[pl.ds(j * _L, _L)] = rg[j]

            o = [oT_v[d, pl.ds(g16, _L)] for d in range(D)]
            ro = [jnp.full((_L,), float(j), jnp.float32) for j in range(D)]
            for a in range(D):
                for c in range(a + 1, D):
                    t = (o[c] > o[a]).astype(jnp.float32)
                    ro[a] = ro[a] + t
                    ro[c] = ro[c] - t

            for j in range(D):
                rgj = rgt_v[pl.ds(j * _L, _L)]
                w = jnp.exp(-rgj)
                dif = ro[j] - rgj - o[j]
                acc = acc + jnp.maximum(dif, 0.0) * w
            return acc

        acc = lax.fori_loop(0, groups, group_body,
                            jnp.zeros((_L,), jnp.float32))
        acc_v[...] = acc
        pltpu.sync_copy(acc_v, part_hbm.at[wid])

    return sc_kernel(out_T, gt_T)


def kernel(batch, gt, W, b):
    Bn, D = batch.shape
    n_tiles = 32
    rows_per_tile = Bn // n_tiles
    out_T, gt_T = _tc_linear_t(batch, gt, W, b)
    parts = _sc_partials(out_T, gt_T, n_tiles, rows_per_tile)
    return jnp.sum(parts) * (1.0 / Bn)
